# baseline (speedup 1.0000x reference)
import functools
import math

import jax
import jax.numpy as jnp
from jax.experimental import pallas as pl
from jax.experimental.pallas import tpu as pltpu

DIM = 384
NUM_HEADS = 12
HD = DIM // NUM_HEADS
KER = 7
NUM_EXPERTS = 8
HID = 768
B, H, W = 2, 32, 32
T = B * H * W
GROUP = 4
NG = H // GROUP
SLAB = GROUP + KER - 1
QBLK = GROUP * W
KVBLK = SLAB * W
SCALE = HD ** -0.5


def _gelu(v):
    return 0.5 * v * (1.0 + jax.lax.erf(v / math.sqrt(2.0)))


def _k1_body(x_ref, g_ref, b_ref, wqkv_ref, bqkv_ref, o_ref):
    x = x_ref[...]
    m = jnp.mean(x, axis=-1, keepdims=True)
    v = jnp.mean((x - m) ** 2, axis=-1, keepdims=True)
    h = (x - m) * jax.lax.rsqrt(v + 1e-5) * g_ref[...] + b_ref[...]
    o_ref[...] = jnp.dot(h, wqkv_ref[...], preferred_element_type=jnp.float32) + bqkv_ref[...]


def _k1(xf, ln1_g, ln1_b, W_qkv, b_qkv):
    blk = 512
    return pl.pallas_call(
        _k1_body,
        grid=(T // blk,),
        in_specs=[
            pl.BlockSpec((blk, DIM), lambda i: (i, 0)),
            pl.BlockSpec((DIM,), lambda i: (0,)),
            pl.BlockSpec((DIM,), lambda i: (0,)),
            pl.BlockSpec((DIM, 3 * DIM), lambda i: (0, 0)),
            pl.BlockSpec((3 * DIM,), lambda i: (0,)),
        ],
        out_specs=pl.BlockSpec((blk, 3 * DIM), lambda i: (i, 0)),
        out_shape=jax.ShapeDtypeStruct((T, 3 * DIM), jnp.float32),
    )(xf, ln1_g, ln1_b, W_qkv, b_qkv)


def _k2_body(bias_ref, q_ref, k_ref, v_ref, o_ref):
    g = pl.program_id(2)
    start = jnp.clip(GROUP * g - (KER // 2), 0, H - SLAB) * W
    q = q_ref[0, 0, 0] * SCALE
    ks = k_ref[0, 0, pl.ds(start, KVBLK), :]
    vs = v_ref[0, 0, pl.ds(start, KVBLK), :]
    s = jax.lax.dot_general(q, ks, (((1,), (1,)), ((), ())),
                            preferred_element_type=jnp.float32)
    s = s + bias_ref[0, 0]
    m = jnp.max(s, axis=-1, keepdims=True)
    p = jnp.exp(s - m)
    p = p / jnp.sum(p, axis=-1, keepdims=True)
    o_ref[0, 0, 0] = jnp.dot(p, vs, preferred_element_type=jnp.float32)


def _k2(bias, q_arr, k_arr, v_arr):
    return pl.pallas_call(
        _k2_body,
        grid=(B, NUM_HEADS, NG),
        in_specs=[
            pl.BlockSpec((1, 1, QBLK, KVBLK), lambda b, n, g: (n, g, 0, 0)),
            pl.BlockSpec((1, 1, 1, QBLK, HD), lambda b, n, g: (b, n, g, 0, 0)),
            pl.BlockSpec((1, 1, H * W, HD), lambda b, n, g: (b, n, 0, 0)),
            pl.BlockSpec((1, 1, H * W, HD), lambda b, n, g: (b, n, 0, 0)),
        ],
        out_specs=pl.BlockSpec((1, 1, 1, QBLK, HD), lambda b, n, g: (b, n, g, 0, 0)),
        out_shape=jax.ShapeDtypeStruct((B, NUM_HEADS, NG, QBLK, HD), jnp.float32),
    )(bias, q_arr, k_arr, v_arr)


def _make_bias(rpb):
    starts = jnp.clip(jnp.arange(H) - KER // 2, 0, H - KER)
    S = jnp.clip(GROUP * jnp.arange(NG) - KER // 2, 0, H - SLAB)
    R = GROUP * jnp.arange(NG)[:, None] + jnp.arange(GROUP)[None, :]
    j = jnp.arange(SLAB)
    ki = S[:, None, None] + j[None, None, :] - starts[R][:, :, None]
    valid_h = (ki >= 0) & (ki < KER)
    rel_h = S[:, None, None] + j[None, None, :] - R[:, :, None] + (KER - 1)
    w2 = jnp.arange(W)
    kj = w2[None, :] - starts[:, None]
    valid_w = (kj >= 0) & (kj < KER)
    rel_w = w2[None, :] - jnp.arange(W)[:, None] + (KER - 1)
    rh = jnp.clip(rel_h, 0, 2 * KER - 2)
    rw = jnp.clip(rel_w, 0, 2 * KER - 2)
    bias = rpb[:, rh[:, :, None, :, None], rw[None, None, :, None, :]]
    valid = valid_h[:, :, None, :, None] & valid_w[None, None, :, None, :]
    bias = jnp.where(valid[None], bias, -1e9)
    return bias.reshape(NUM_HEADS, NG, QBLK, KVBLK)


def _k3_body(attn_ref, x_ref, wp_ref, bp_ref, g2_ref, b2_ref, wr_ref, br_ref,
             ws1_ref, bs1_ref, ws2_ref, bs2_ref,
             y_ref, base_ref, comb_ref):
    x2 = x_ref[...] + jnp.dot(attn_ref[...], wp_ref[...],
                              preferred_element_type=jnp.float32) + bp_ref[...]
    m = jnp.mean(x2, axis=-1, keepdims=True)
    v = jnp.mean((x2 - m) ** 2, axis=-1, keepdims=True)
    y = (x2 - m) * jax.lax.rsqrt(v + 1e-5) * g2_ref[...] + b2_ref[...]
    y_ref[...] = y
    logits = jnp.dot(y, wr_ref[...], preferred_element_type=jnp.float32) + br_ref[...]
    mx = jnp.max(logits, axis=-1, keepdims=True)
    eg = jnp.exp(logits - mx)
    gates = eg / jnp.sum(eg, axis=-1, keepdims=True)
    iota = jax.lax.broadcasted_iota(jnp.int32, gates.shape, 1)
    v1 = jnp.max(gates, axis=-1, keepdims=True)
    i1 = jnp.min(jnp.where(gates == v1, iota, NUM_EXPERTS), axis=-1, keepdims=True)
    m1 = iota == i1
    g2nd = jnp.where(m1, -1.0, gates)
    v2 = jnp.max(g2nd, axis=-1, keepdims=True)
    i2 = jnp.min(jnp.where(g2nd == v2, iota, NUM_EXPERTS), axis=-1, keepdims=True)
    m2 = iota == i2
    denom = v1 + v2
    comb = jnp.where(m1, v1, 0.0) + jnp.where(m2, v2, 0.0)
    comb_ref[...] = jnp.transpose(comb / denom, (1, 0))
    hsh = _gelu(jnp.dot(y, ws1_ref[...], preferred_element_type=jnp.float32) + bs1_ref[...])
    shared = jnp.dot(hsh, ws2_ref[...], preferred_element_type=jnp.float32) + bs2_ref[...]
    base_ref[...] = x2 + shared


def _k3(attn, xf, W_proj, b_proj, ln2_g, ln2_b, W_r, b_r, W_s1, b_s1, W_s2, b_s2):
    full = lambda *shape: pl.BlockSpec(shape, lambda: tuple(0 for _ in shape))
    return pl.pallas_call(
        _k3_body,
        grid=(),
        in_specs=[
            full(T, DIM), full(T, DIM), full(DIM, DIM), full(DIM),
            full(DIM), full(DIM), full(DIM, NUM_EXPERTS), full(NUM_EXPERTS),
            full(DIM, HID), full(HID), full(HID, DIM), full(DIM),
        ],
        out_specs=[full(T, DIM), full(T, DIM), full(NUM_EXPERTS, T)],
        out_shape=[
            jax.ShapeDtypeStruct((T, DIM), jnp.float32),
            jax.ShapeDtypeStruct((T, DIM), jnp.float32),
            jax.ShapeDtypeStruct((NUM_EXPERTS, T), jnp.float32),
        ],
    )(attn, xf, W_proj, b_proj, ln2_g, ln2_b, W_r, b_r, W_s1, b_s1, W_s2, b_s2)


def _k4_body(y_ref, comb_ref, w1_ref, b1_ref, w2_ref, b2_ref, base_ref, o_ref):
    e = pl.program_id(0)
    h = _gelu(jnp.dot(y_ref[...], w1_ref[0], preferred_element_type=jnp.float32) + b1_ref[0])
    eo = jnp.dot(h, w2_ref[0], preferred_element_type=jnp.float32) + b2_ref[0]
    contrib = comb_ref[0] * eo

    @pl.when(e == 0)
    def _():
        o_ref[...] = base_ref[...] + contrib

    @pl.when(e > 0)
    def _():
        o_ref[...] = o_ref[...] + contrib


def _k4(y, comb3, W_e1, b_e1, W_e2, b_e2, base):
    return pl.pallas_call(
        _k4_body,
        grid=(NUM_EXPERTS,),
        in_specs=[
            pl.BlockSpec((T, DIM), lambda e: (0, 0)),
            pl.BlockSpec((1, T, 1), lambda e: (e, 0, 0)),
            pl.BlockSpec((1, DIM, HID), lambda e: (e, 0, 0)),
            pl.BlockSpec((1, 1, HID), lambda e: (e, 0, 0)),
            pl.BlockSpec((1, HID, DIM), lambda e: (e, 0, 0)),
            pl.BlockSpec((1, 1, DIM), lambda e: (e, 0, 0)),
            pl.BlockSpec((T, DIM), lambda e: (0, 0)),
        ],
        out_specs=pl.BlockSpec((T, DIM), lambda e: (0, 0)),
        out_shape=jax.ShapeDtypeStruct((T, DIM), jnp.float32),
    )(y, comb3, W_e1, b_e1, W_e2, b_e2, base)


def kernel(x, ln1_g, ln1_b, ln2_g, ln2_b, W_qkv, b_qkv, rpb, W_proj, b_proj,
           W_r, b_r, W_e1, b_e1, W_e2, b_e2, W_s1, b_s1, W_s2, b_s2):
    xf = x.reshape(T, DIM)
    qkv = _k1(xf, ln1_g, ln1_b, W_qkv, b_qkv)
    qkv6 = qkv.reshape(B, H, W, 3, NUM_HEADS, HD)
    q = jnp.transpose(qkv6[:, :, :, 0], (0, 3, 1, 2, 4))
    k = jnp.transpose(qkv6[:, :, :, 1], (0, 3, 1, 2, 4))
    v = jnp.transpose(qkv6[:, :, :, 2], (0, 3, 1, 2, 4))
    q_arr = q.reshape(B, NUM_HEADS, NG, QBLK, HD)
    k_arr = k.reshape(B, NUM_HEADS, H * W, HD)
    v_arr = v.reshape(B, NUM_HEADS, H * W, HD)
    bias = _make_bias(rpb)
    o = _k2(bias, q_arr, k_arr, v_arr)
    attn = jnp.transpose(
        o.reshape(B, NUM_HEADS, NG, GROUP, W, HD), (0, 2, 3, 4, 1, 5)
    ).reshape(T, DIM)
    y, base, comb = _k3(attn, xf, W_proj, b_proj, ln2_g, ln2_b,
                        W_r, b_r, W_s1, b_s1, W_s2, b_s2)
    comb3 = comb.reshape(NUM_EXPERTS, T, 1)
    out = _k4(y, comb3, W_e1, b_e1.reshape(NUM_EXPERTS, 1, HID),
              W_e2, b_e2.reshape(NUM_EXPERTS, 1, DIM), base)
    return out.reshape(B, H, W, DIM)


# speedup vs baseline: 1.0290x; 1.0290x over previous
import functools
import math

import jax
import jax.numpy as jnp
from jax.experimental import pallas as pl
from jax.experimental.pallas import tpu as pltpu

DIM = 384
NUM_HEADS = 12
HD = DIM // NUM_HEADS
KER = 7
NUM_EXPERTS = 8
HID = 768
B, H, W = 2, 32, 32
T = B * H * W
GROUP = 4
NG = H // GROUP
SLAB = GROUP + KER - 1
QBLK = GROUP * W
KVBLK = SLAB * W
SCALE = HD ** -0.5
BF = jnp.bfloat16


def _gelu(v):
    return 0.5 * v * (1.0 + jax.lax.erf(v / math.sqrt(2.0)))


def _dotf32(a, b, trans_b=False):
    dn = (((1,), (1,)), ((), ())) if trans_b else (((1,), (0,)), ((), ()))
    return jax.lax.dot_general(a, b, dn, preferred_element_type=jnp.float32)


def _k1_body(x_ref, g_ref, b_ref, wqkv_ref, bqkv_ref, o_ref):
    x = x_ref[...]
    m = jnp.mean(x, axis=-1, keepdims=True)
    v = jnp.mean((x - m) ** 2, axis=-1, keepdims=True)
    h = (x - m) * jax.lax.rsqrt(v + 1e-5) * g_ref[...] + b_ref[...]
    acc = _dotf32(h.astype(BF), wqkv_ref[...]) + bqkv_ref[...]
    o_ref[...] = acc.astype(BF)


def _k1(xf, ln1_g, ln1_b, W_qkv_bf, b_qkv):
    blk = 512
    return pl.pallas_call(
        _k1_body,
        grid=(T // blk,),
        in_specs=[
            pl.BlockSpec((blk, DIM), lambda i: (i, 0)),
            pl.BlockSpec((DIM,), lambda i: (0,)),
            pl.BlockSpec((DIM,), lambda i: (0,)),
            pl.BlockSpec((DIM, 3 * DIM), lambda i: (0, 0)),
            pl.BlockSpec((3 * DIM,), lambda i: (0,)),
        ],
        out_specs=pl.BlockSpec((blk, 3 * DIM), lambda i: (i, 0)),
        out_shape=jax.ShapeDtypeStruct((T, 3 * DIM), BF),
    )(xf, ln1_g, ln1_b, W_qkv_bf, b_qkv)


def _k2_body(bias_ref, q_ref, kv_ref, o_ref):
    g = pl.program_id(1)
    start = jnp.clip(GROUP * g - (KER // 2), 0, H - SLAB) * W
    for n in range(NUM_HEADS):
        q = q_ref[0, :, n * HD:(n + 1) * HD]
        ks = kv_ref[0, pl.ds(start, KVBLK), DIM + n * HD:DIM + (n + 1) * HD]
        vs = kv_ref[0, pl.ds(start, KVBLK), 2 * DIM + n * HD:2 * DIM + (n + 1) * HD]
        s = _dotf32(q, ks, trans_b=True) * SCALE + bias_ref[n, 0]
        mx = jnp.max(s, axis=-1, keepdims=True)
        p = jnp.exp(s - mx)
        p = p / jnp.sum(p, axis=-1, keepdims=True)
        o_ref[0, :, n * HD:(n + 1) * HD] = _dotf32(p.astype(BF), vs)


def _k2(bias, qkv3):
    return pl.pallas_call(
        _k2_body,
        grid=(B, NG),
        in_specs=[
            pl.BlockSpec((NUM_HEADS, 1, QBLK, KVBLK), lambda b, g: (0, g, 0, 0)),
            pl.BlockSpec((1, QBLK, 3 * DIM), lambda b, g: (b, g, 0)),
            pl.BlockSpec((1, H * W, 3 * DIM), lambda b, g: (b, 0, 0)),
        ],
        out_specs=pl.BlockSpec((1, QBLK, DIM), lambda b, g: (b, g, 0)),
        out_shape=jax.ShapeDtypeStruct((B, H * W, DIM), jnp.float32),
    )(bias, qkv3, qkv3)


def _make_bias(rpb):
    starts = jnp.clip(jnp.arange(H) - KER // 2, 0, H - KER)
    S = jnp.clip(GROUP * jnp.arange(NG) - KER // 2, 0, H - SLAB)
    R = GROUP * jnp.arange(NG)[:, None] + jnp.arange(GROUP)[None, :]
    j = jnp.arange(SLAB)
    ki = S[:, None, None] + j[None, None, :] - starts[R][:, :, None]
    valid_h = (ki >= 0) & (ki < KER)
    rel_h = S[:, None, None] + j[None, None, :] - R[:, :, None] + (KER - 1)
    w2 = jnp.arange(W)
    kj = w2[None, :] - starts[:, None]
    valid_w = (kj >= 0) & (kj < KER)
    rel_w = w2[None, :] - jnp.arange(W)[:, None] + (KER - 1)
    rh = jnp.clip(rel_h, 0, 2 * KER - 2)
    rw = jnp.clip(rel_w, 0, 2 * KER - 2)
    bias = rpb[:, rh[:, :, None, :, None], rw[None, None, :, None, :]]
    valid = valid_h[:, :, None, :, None] & valid_w[None, None, :, None, :]
    bias = jnp.where(valid[None], bias, -1e9)
    return bias.reshape(NUM_HEADS, NG, QBLK, KVBLK)


def _k3_body(attn_ref, x_ref, wp_ref, bp_ref, g2_ref, b2_ref, wr_ref, br_ref,
             ws1_ref, bs1_ref, ws2_ref, bs2_ref,
             y_ref, base_ref, comb_ref):
    x2 = x_ref[...] + _dotf32(attn_ref[...].astype(BF), wp_ref[...]) + bp_ref[...]
    m = jnp.mean(x2, axis=-1, keepdims=True)
    v = jnp.mean((x2 - m) ** 2, axis=-1, keepdims=True)
    y = (x2 - m) * jax.lax.rsqrt(v + 1e-5) * g2_ref[...] + b2_ref[...]
    yb = y.astype(BF)
    y_ref[...] = yb
    logits = _dotf32(y, wr_ref[...]) + br_ref[...]
    mx = jnp.max(logits, axis=-1, keepdims=True)
    eg = jnp.exp(logits - mx)
    gates = eg / jnp.sum(eg, axis=-1, keepdims=True)
    iota = jax.lax.broadcasted_iota(jnp.int32, gates.shape, 1)
    v1 = jnp.max(gates, axis=-1, keepdims=True)
    i1 = jnp.min(jnp.where(gates == v1, iota, NUM_EXPERTS), axis=-1, keepdims=True)
    m1 = iota == i1
    g2nd = jnp.where(m1, -1.0, gates)
    v2 = jnp.max(g2nd, axis=-1, keepdims=True)
    i2 = jnp.min(jnp.where(g2nd == v2, iota, NUM_EXPERTS), axis=-1, keepdims=True)
    m2 = iota == i2
    denom = v1 + v2
    comb = jnp.where(m1, v1, 0.0) + jnp.where(m2, v2, 0.0)
    comb_ref[...] = jnp.transpose(comb / denom, (1, 0))
    hsh = _gelu(_dotf32(yb, ws1_ref[...]) + bs1_ref[...])
    shared = _dotf32(hsh.astype(BF), ws2_ref[...]) + bs2_ref[...]
    base_ref[...] = x2 + shared


def _k3(attn, xf, Wp_bf, b_proj, ln2_g, ln2_b, W_r, b_r, Ws1_bf, b_s1, Ws2_bf, b_s2):
    blk = 512
    return pl.pallas_call(
        _k3_body,
        grid=(T // blk,),
        in_specs=[
            pl.BlockSpec((blk, DIM), lambda i: (i, 0)),
            pl.BlockSpec((blk, DIM), lambda i: (i, 0)),
            pl.BlockSpec((DIM, DIM), lambda i: (0, 0)),
            pl.BlockSpec((DIM,), lambda i: (0,)),
            pl.BlockSpec((DIM,), lambda i: (0,)),
            pl.BlockSpec((DIM,), lambda i: (0,)),
            pl.BlockSpec((DIM, NUM_EXPERTS), lambda i: (0, 0)),
            pl.BlockSpec((NUM_EXPERTS,), lambda i: (0,)),
            pl.BlockSpec((DIM, HID), lambda i: (0, 0)),
            pl.BlockSpec((HID,), lambda i: (0,)),
            pl.BlockSpec((HID, DIM), lambda i: (0, 0)),
            pl.BlockSpec((DIM,), lambda i: (0,)),
        ],
        out_specs=[
            pl.BlockSpec((blk, DIM), lambda i: (i, 0)),
            pl.BlockSpec((blk, DIM), lambda i: (i, 0)),
            pl.BlockSpec((NUM_EXPERTS, blk), lambda i: (0, i)),
        ],
        out_shape=[
            jax.ShapeDtypeStruct((T, DIM), BF),
            jax.ShapeDtypeStruct((T, DIM), jnp.float32),
            jax.ShapeDtypeStruct((NUM_EXPERTS, T), jnp.float32),
        ],
    )(attn, xf, Wp_bf, b_proj, ln2_g, ln2_b, W_r, b_r, Ws1_bf, b_s1, Ws2_bf, b_s2)


def _k4_body(y_ref, comb_ref, w1_ref, b1_ref, w2_ref, b2_ref, base_ref, o_ref):
    e = pl.program_id(0)
    h = _gelu(_dotf32(y_ref[...], w1_ref[0]) + b1_ref[0])
    eo = _dotf32(h.astype(BF), w2_ref[0]) + b2_ref[0]
    contrib = comb_ref[0] * eo

    @pl.when(e == 0)
    def _():
        o_ref[...] = base_ref[...] + contrib

    @pl.when(e > 0)
    def _():
        o_ref[...] = o_ref[...] + contrib


def _k4(y_bf, comb3, We1_bf, b_e1, We2_bf, b_e2, base):
    return pl.pallas_call(
        _k4_body,
        grid=(NUM_EXPERTS,),
        in_specs=[
            pl.BlockSpec((T, DIM), lambda e: (0, 0)),
            pl.BlockSpec((1, T, 1), lambda e: (e, 0, 0)),
            pl.BlockSpec((1, DIM, HID), lambda e: (e, 0, 0)),
            pl.BlockSpec((1, 1, HID), lambda e: (e, 0, 0)),
            pl.BlockSpec((1, HID, DIM), lambda e: (e, 0, 0)),
            pl.BlockSpec((1, 1, DIM), lambda e: (e, 0, 0)),
            pl.BlockSpec((T, DIM), lambda e: (0, 0)),
        ],
        out_specs=pl.BlockSpec((T, DIM), lambda e: (0, 0)),
        out_shape=jax.ShapeDtypeStruct((T, DIM), jnp.float32),
    )(y_bf, comb3, We1_bf, b_e1, We2_bf, b_e2, base)


def kernel(x, ln1_g, ln1_b, ln2_g, ln2_b, W_qkv, b_qkv, rpb, W_proj, b_proj,
           W_r, b_r, W_e1, b_e1, W_e2, b_e2, W_s1, b_s1, W_s2, b_s2):
    xf = x.reshape(T, DIM)
    qkv = _k1(xf, ln1_g, ln1_b, W_qkv.astype(BF), b_qkv)
    qkv3 = qkv.reshape(B, H * W, 3 * DIM)
    bias = _make_bias(rpb)
    attn = _k2(bias, qkv3).reshape(T, DIM)
    y_bf, base, comb = _k3(attn, xf, W_proj.astype(BF), b_proj, ln2_g, ln2_b,
                           W_r, b_r, W_s1.astype(BF), b_s1, W_s2.astype(BF), b_s2)
    comb3 = comb.reshape(NUM_EXPERTS, T, 1)
    out = _k4(y_bf, comb3, W_e1.astype(BF), b_e1.reshape(NUM_EXPERTS, 1, HID),
              W_e2.astype(BF), b_e2.reshape(NUM_EXPERTS, 1, DIM), base)
    return out.reshape(B, H, W, DIM)


# speedup vs baseline: 16.9183x; 16.4412x over previous
import functools
import math

import jax
import jax.numpy as jnp
from jax.experimental import pallas as pl
from jax.experimental.pallas import tpu as pltpu

DIM = 384
NUM_HEADS = 12
HD = DIM // NUM_HEADS
KER = 7
NUM_EXPERTS = 8
HID = 768
B, H, W = 2, 32, 32
T = B * H * W
GROUP = 4
NG = H // GROUP
SLAB = GROUP + KER - 1
QBLK = GROUP * W
KVBLK = SLAB * W
SCALE = HD ** -0.5
BF = jnp.bfloat16


def _gelu(v):
    return 0.5 * v * (1.0 + jax.lax.erf(v / math.sqrt(2.0)))


def _dotf32(a, b, trans_b=False):
    dn = (((1,), (1,)), ((), ())) if trans_b else (((1,), (0,)), ((), ()))
    return jax.lax.dot_general(a, b, dn, preferred_element_type=jnp.float32)


def _k1_body(x_ref, g_ref, b_ref, wqkv_ref, bqkv_ref, o_ref):
    x = x_ref[...]
    m = jnp.mean(x, axis=-1, keepdims=True)
    v = jnp.mean((x - m) ** 2, axis=-1, keepdims=True)
    h = (x - m) * jax.lax.rsqrt(v + 1e-5) * g_ref[...] + b_ref[...]
    acc = _dotf32(h.astype(BF), wqkv_ref[...]) + bqkv_ref[...]
    o_ref[...] = acc.astype(BF)


def _k1(xf, ln1_g, ln1_b, W_qkv_bf, b_qkv):
    blk = 512
    return pl.pallas_call(
        _k1_body,
        grid=(T // blk,),
        in_specs=[
            pl.BlockSpec((blk, DIM), lambda i: (i, 0)),
            pl.BlockSpec((DIM,), lambda i: (0,)),
            pl.BlockSpec((DIM,), lambda i: (0,)),
            pl.BlockSpec((DIM, 3 * DIM), lambda i: (0, 0)),
            pl.BlockSpec((3 * DIM,), lambda i: (0,)),
        ],
        out_specs=pl.BlockSpec((blk, 3 * DIM), lambda i: (i, 0)),
        out_shape=jax.ShapeDtypeStruct((T, 3 * DIM), BF),
    )(xf, ln1_g, ln1_b, W_qkv_bf, b_qkv)


def _k2_body(bias_ref, q_ref, kv_ref, o_ref):
    g = pl.program_id(1)
    start = jnp.clip(GROUP * g - (KER // 2), 0, H - SLAB) * W
    for n in range(NUM_HEADS):
        q = q_ref[0, :, n * HD:(n + 1) * HD]
        ks = kv_ref[0, pl.ds(start, KVBLK), DIM + n * HD:DIM + (n + 1) * HD]
        vs = kv_ref[0, pl.ds(start, KVBLK), 2 * DIM + n * HD:2 * DIM + (n + 1) * HD]
        s = _dotf32(q, ks, trans_b=True) * SCALE + bias_ref[n, 0]
        mx = jnp.max(s, axis=-1, keepdims=True)
        p = jnp.exp(s - mx)
        p = p / jnp.sum(p, axis=-1, keepdims=True)
        o_ref[0, :, n * HD:(n + 1) * HD] = _dotf32(p.astype(BF), vs)


def _k2(bias, qkv3):
    return pl.pallas_call(
        _k2_body,
        grid=(B, NG),
        in_specs=[
            pl.BlockSpec((NUM_HEADS, 1, QBLK, KVBLK), lambda b, g: (0, g, 0, 0)),
            pl.BlockSpec((1, QBLK, 3 * DIM), lambda b, g: (b, g, 0)),
            pl.BlockSpec((1, H * W, 3 * DIM), lambda b, g: (b, 0, 0)),
        ],
        out_specs=pl.BlockSpec((1, QBLK, DIM), lambda b, g: (b, g, 0)),
        out_shape=jax.ShapeDtypeStruct((B, H * W, DIM), jnp.float32),
    )(bias, qkv3, qkv3)


def _bias_tables():
    import numpy as np
    NR = 2 * KER - 1
    starts = np.clip(np.arange(H) - KER // 2, 0, H - KER)
    S = np.clip(GROUP * np.arange(NG) - KER // 2, 0, H - SLAB)
    R = GROUP * np.arange(NG)[:, None] + np.arange(GROUP)[None, :]
    j = np.arange(SLAB)
    ki = S[:, None, None] + j[None, None, :] - starts[R][:, :, None]
    valid_h = (ki >= 0) & (ki < KER)
    rel_h = S[:, None, None] + j[None, None, :] - R[:, :, None] + (KER - 1)
    oh_h = np.eye(NR, dtype=np.float32)[np.clip(rel_h, 0, NR - 1)] * valid_h[..., None]
    w2 = np.arange(W)
    kj = w2[None, :] - starts[:, None]
    valid_w = (kj >= 0) & (kj < KER)
    rel_w = w2[None, :] - np.arange(W)[:, None] + (KER - 1)
    oh_w = np.eye(NR, dtype=np.float32)[np.clip(rel_w, 0, NR - 1)] * valid_w[..., None]
    valid = valid_h[:, :, None, :, None] & valid_w[None, None, :, None, :]
    mask = np.where(valid, 0.0, -1e9).astype(np.float32)
    return oh_h, oh_w, mask


_OH_H, _OH_W, _MASK = _bias_tables()


def _make_bias(rpb):
    t = jnp.einsum('grja,nab->ngrjb', jnp.asarray(_OH_H), rpb)
    bias = jnp.einsum('ngrjb,wvb->ngrwjv', t, jnp.asarray(_OH_W))
    bias = bias + jnp.asarray(_MASK)[None]
    return bias.reshape(NUM_HEADS, NG, QBLK, KVBLK)


def _k3_body(attn_ref, x_ref, wp_ref, bp_ref, g2_ref, b2_ref, wr_ref, br_ref,
             ws1_ref, bs1_ref, ws2_ref, bs2_ref,
             y_ref, base_ref, comb_ref):
    x2 = x_ref[...] + _dotf32(attn_ref[...].astype(BF), wp_ref[...]) + bp_ref[...]
    m = jnp.mean(x2, axis=-1, keepdims=True)
    v = jnp.mean((x2 - m) ** 2, axis=-1, keepdims=True)
    y = (x2 - m) * jax.lax.rsqrt(v + 1e-5) * g2_ref[...] + b2_ref[...]
    yb = y.astype(BF)
    y_ref[...] = yb
    logits = _dotf32(y, wr_ref[...]) + br_ref[...]
    mx = jnp.max(logits, axis=-1, keepdims=True)
    eg = jnp.exp(logits - mx)
    gates = eg / jnp.sum(eg, axis=-1, keepdims=True)
    iota = jax.lax.broadcasted_iota(jnp.int32, gates.shape, 1)
    v1 = jnp.max(gates, axis=-1, keepdims=True)
    i1 = jnp.min(jnp.where(gates == v1, iota, NUM_EXPERTS), axis=-1, keepdims=True)
    m1 = iota == i1
    g2nd = jnp.where(m1, -1.0, gates)
    v2 = jnp.max(g2nd, axis=-1, keepdims=True)
    i2 = jnp.min(jnp.where(g2nd == v2, iota, NUM_EXPERTS), axis=-1, keepdims=True)
    m2 = iota == i2
    denom = v1 + v2
    comb = jnp.where(m1, v1, 0.0) + jnp.where(m2, v2, 0.0)
    comb_ref[...] = jnp.transpose(comb / denom, (1, 0))
    hsh = _gelu(_dotf32(yb, ws1_ref[...]) + bs1_ref[...])
    shared = _dotf32(hsh.astype(BF), ws2_ref[...]) + bs2_ref[...]
    base_ref[...] = x2 + shared


def _k3(attn, xf, Wp_bf, b_proj, ln2_g, ln2_b, W_r, b_r, Ws1_bf, b_s1, Ws2_bf, b_s2):
    blk = 512
    return pl.pallas_call(
        _k3_body,
        grid=(T // blk,),
        in_specs=[
            pl.BlockSpec((blk, DIM), lambda i: (i, 0)),
            pl.BlockSpec((blk, DIM), lambda i: (i, 0)),
            pl.BlockSpec((DIM, DIM), lambda i: (0, 0)),
            pl.BlockSpec((DIM,), lambda i: (0,)),
            pl.BlockSpec((DIM,), lambda i: (0,)),
            pl.BlockSpec((DIM,), lambda i: (0,)),
            pl.BlockSpec((DIM, NUM_EXPERTS), lambda i: (0, 0)),
            pl.BlockSpec((NUM_EXPERTS,), lambda i: (0,)),
            pl.BlockSpec((DIM, HID), lambda i: (0, 0)),
            pl.BlockSpec((HID,), lambda i: (0,)),
            pl.BlockSpec((HID, DIM), lambda i: (0, 0)),
            pl.BlockSpec((DIM,), lambda i: (0,)),
        ],
        out_specs=[
            pl.BlockSpec((blk, DIM), lambda i: (i, 0)),
            pl.BlockSpec((blk, DIM), lambda i: (i, 0)),
            pl.BlockSpec((NUM_EXPERTS, blk), lambda i: (0, i)),
        ],
        out_shape=[
            jax.ShapeDtypeStruct((T, DIM), BF),
            jax.ShapeDtypeStruct((T, DIM), jnp.float32),
            jax.ShapeDtypeStruct((NUM_EXPERTS, T), jnp.float32),
        ],
    )(attn, xf, Wp_bf, b_proj, ln2_g, ln2_b, W_r, b_r, Ws1_bf, b_s1, Ws2_bf, b_s2)


def _k4_body(y_ref, comb_ref, w1_ref, b1_ref, w2_ref, b2_ref, base_ref, o_ref):
    e = pl.program_id(0)
    h = _gelu(_dotf32(y_ref[...], w1_ref[0]) + b1_ref[0])
    eo = _dotf32(h.astype(BF), w2_ref[0]) + b2_ref[0]
    contrib = comb_ref[0] * eo

    @pl.when(e == 0)
    def _():
        o_ref[...] = base_ref[...] + contrib

    @pl.when(e > 0)
    def _():
        o_ref[...] = o_ref[...] + contrib


def _k4(y_bf, comb3, We1_bf, b_e1, We2_bf, b_e2, base):
    return pl.pallas_call(
        _k4_body,
        grid=(NUM_EXPERTS,),
        in_specs=[
            pl.BlockSpec((T, DIM), lambda e: (0, 0)),
            pl.BlockSpec((1, T, 1), lambda e: (e, 0, 0)),
            pl.BlockSpec((1, DIM, HID), lambda e: (e, 0, 0)),
            pl.BlockSpec((1, 1, HID), lambda e: (e, 0, 0)),
            pl.BlockSpec((1, HID, DIM), lambda e: (e, 0, 0)),
            pl.BlockSpec((1, 1, DIM), lambda e: (e, 0, 0)),
            pl.BlockSpec((T, DIM), lambda e: (0, 0)),
        ],
        out_specs=pl.BlockSpec((T, DIM), lambda e: (0, 0)),
        out_shape=jax.ShapeDtypeStruct((T, DIM), jnp.float32),
    )(y_bf, comb3, We1_bf, b_e1, We2_bf, b_e2, base)


def kernel(x, ln1_g, ln1_b, ln2_g, ln2_b, W_qkv, b_qkv, rpb, W_proj, b_proj,
           W_r, b_r, W_e1, b_e1, W_e2, b_e2, W_s1, b_s1, W_s2, b_s2):
    xf = x.reshape(T, DIM)
    qkv = _k1(xf, ln1_g, ln1_b, W_qkv.astype(BF), b_qkv)
    qkv3 = qkv.reshape(B, H * W, 3 * DIM)
    bias = _make_bias(rpb)
    attn = _k2(bias, qkv3).reshape(T, DIM)
    y_bf, base, comb = _k3(attn, xf, W_proj.astype(BF), b_proj, ln2_g, ln2_b,
                           W_r, b_r, W_s1.astype(BF), b_s1, W_s2.astype(BF), b_s2)
    comb3 = comb.reshape(NUM_EXPERTS, T, 1)
    out = _k4(y_bf, comb3, W_e1.astype(BF), b_e1.reshape(NUM_EXPERTS, 1, HID),
              W_e2.astype(BF), b_e2.reshape(NUM_EXPERTS, 1, DIM), base)
    return out.reshape(B, H, W, DIM)


# speedup vs baseline: 18.3148x; 1.0825x over previous
import functools
import math

import jax
import jax.numpy as jnp
from jax.experimental import pallas as pl
from jax.experimental.pallas import tpu as pltpu

DIM = 384
NUM_HEADS = 12
HD = DIM // NUM_HEADS
KER = 7
NUM_EXPERTS = 8
HID = 768
B, H, W = 2, 32, 32
T = B * H * W
GROUP = 4
NG = H // GROUP
SLAB = GROUP + KER - 1
QBLK = GROUP * W
KVBLK = SLAB * W
SCALE = HD ** -0.5
BF = jnp.bfloat16


def _gelu(v):
    return 0.5 * v * (1.0 + jax.lax.erf(v / math.sqrt(2.0)))


def _dotf32(a, b, trans_b=False):
    dn = (((1,), (1,)), ((), ())) if trans_b else (((1,), (0,)), ((), ()))
    return jax.lax.dot_general(a, b, dn, preferred_element_type=jnp.float32)


def _k1_body(x_ref, g_ref, b_ref, wqkv_ref, bqkv_ref, o_ref):
    x = x_ref[...]
    m = jnp.mean(x, axis=-1, keepdims=True)
    v = jnp.mean((x - m) ** 2, axis=-1, keepdims=True)
    h = (x - m) * jax.lax.rsqrt(v + 1e-5) * g_ref[...] + b_ref[...]
    acc = _dotf32(h.astype(BF), wqkv_ref[...]) + bqkv_ref[...]
    o_ref[...] = acc.astype(BF)


def _k1(xf, ln1_g, ln1_b, W_qkv_bf, b_qkv):
    blk = 512
    return pl.pallas_call(
        _k1_body,
        grid=(T // blk,),
        in_specs=[
            pl.BlockSpec((blk, DIM), lambda i: (i, 0)),
            pl.BlockSpec((DIM,), lambda i: (0,)),
            pl.BlockSpec((DIM,), lambda i: (0,)),
            pl.BlockSpec((DIM, 3 * DIM), lambda i: (0, 0)),
            pl.BlockSpec((3 * DIM,), lambda i: (0,)),
        ],
        out_specs=pl.BlockSpec((blk, 3 * DIM), lambda i: (i, 0)),
        out_shape=jax.ShapeDtypeStruct((T, 3 * DIM), BF),
    )(xf, ln1_g, ln1_b, W_qkv_bf, b_qkv)


def _k2_body(bias_ref, q_ref, kv_ref, o_ref):
    g = pl.program_id(1)
    start = jnp.clip(GROUP * g - (KER // 2), 0, H - SLAB) * W
    w_q = jax.lax.broadcasted_iota(jnp.int32, (QBLK, KVBLK), 0) & (W - 1)
    w_k = jax.lax.broadcasted_iota(jnp.int32, (QBLK, KVBLK), 1) & (W - 1)
    kj = w_k - jnp.clip(w_q - KER // 2, 0, W - KER)
    maskw = jnp.where((kj >= 0) & (kj < KER), 0.0, -1e9)
    for n in range(NUM_HEADS):
        q = q_ref[0, :, n * HD:(n + 1) * HD]
        ks = kv_ref[0, pl.ds(start, KVBLK), DIM + n * HD:DIM + (n + 1) * HD]
        vs = kv_ref[0, pl.ds(start, KVBLK), 2 * DIM + n * HD:2 * DIM + (n + 1) * HD]
        bias_n = jnp.concatenate(
            [bias_ref[n, 0, :, j].reshape(QBLK, W) for j in range(SLAB)], axis=1)
        s = _dotf32(q, ks, trans_b=True) * SCALE + bias_n + maskw
        mx = jnp.max(s, axis=-1, keepdims=True)
        p = jnp.exp(s - mx)
        p = p / jnp.sum(p, axis=-1, keepdims=True)
        o_ref[0, :, n * HD:(n + 1) * HD] = _dotf32(p.astype(BF), vs)


def _k2(bias, qkv3):
    return pl.pallas_call(
        _k2_body,
        grid=(B, NG),
        in_specs=[
            pl.BlockSpec((NUM_HEADS, 1, GROUP, SLAB, W, W),
                         lambda b, g: (0, g, 0, 0, 0, 0)),
            pl.BlockSpec((1, QBLK, 3 * DIM), lambda b, g: (b, g, 0)),
            pl.BlockSpec((1, H * W, 3 * DIM), lambda b, g: (b, 0, 0)),
        ],
        out_specs=pl.BlockSpec((1, QBLK, DIM), lambda b, g: (b, g, 0)),
        out_shape=jax.ShapeDtypeStruct((B, H * W, DIM), jnp.float32),
    )(bias, qkv3, qkv3)


def _bias_tables():
    import numpy as np
    NR = 2 * KER - 1
    starts = np.clip(np.arange(H) - KER // 2, 0, H - KER)
    S = np.clip(GROUP * np.arange(NG) - KER // 2, 0, H - SLAB)
    R = GROUP * np.arange(NG)[:, None] + np.arange(GROUP)[None, :]
    j = np.arange(SLAB)
    ki = S[:, None, None] + j[None, None, :] - starts[R][:, :, None]
    valid_h = (ki >= 0) & (ki < KER)
    rel_h = S[:, None, None] + j[None, None, :] - R[:, :, None] + (KER - 1)
    oh_h = np.eye(NR, dtype=np.float32)[np.clip(rel_h, 0, NR - 1)] * valid_h[..., None]
    maskh = np.where(valid_h, 0.0, -1e9).astype(np.float32)
    w2 = np.arange(W)
    kj = w2[None, :] - starts[:, None]
    valid_w = (kj >= 0) & (kj < KER)
    rel_w = w2[None, :] - np.arange(W)[:, None] + (KER - 1)
    oh_w = np.eye(NR, dtype=np.float32)[np.clip(rel_w, 0, NR - 1)] * valid_w[..., None]
    return oh_h, oh_w, maskh


_OH_H, _OH_W, _MASKH = _bias_tables()


def _make_bias(rpb):
    t = jnp.einsum('grja,nab->ngrjb', jnp.asarray(_OH_H), rpb)
    t = t + jnp.asarray(_MASKH)[None, :, :, :, None]
    bias = jnp.einsum('ngrjb,wvb->ngrjwv', t, jnp.asarray(_OH_W))
    return bias


def _k3_body(attn_ref, x_ref, wp_ref, bp_ref, g2_ref, b2_ref, wr_ref, br_ref,
             ws1_ref, bs1_ref, ws2_ref, bs2_ref,
             y_ref, base_ref, comb_ref):
    x2 = x_ref[...] + _dotf32(attn_ref[...].astype(BF), wp_ref[...]) + bp_ref[...]
    m = jnp.mean(x2, axis=-1, keepdims=True)
    v = jnp.mean((x2 - m) ** 2, axis=-1, keepdims=True)
    y = (x2 - m) * jax.lax.rsqrt(v + 1e-5) * g2_ref[...] + b2_ref[...]
    yb = y.astype(BF)
    y_ref[...] = yb
    logits = _dotf32(y, wr_ref[...]) + br_ref[...]
    mx = jnp.max(logits, axis=-1, keepdims=True)
    eg = jnp.exp(logits - mx)
    gates = eg / jnp.sum(eg, axis=-1, keepdims=True)
    iota = jax.lax.broadcasted_iota(jnp.int32, gates.shape, 1)
    v1 = jnp.max(gates, axis=-1, keepdims=True)
    i1 = jnp.min(jnp.where(gates == v1, iota, NUM_EXPERTS), axis=-1, keepdims=True)
    m1 = iota == i1
    g2nd = jnp.where(m1, -1.0, gates)
    v2 = jnp.max(g2nd, axis=-1, keepdims=True)
    i2 = jnp.min(jnp.where(g2nd == v2, iota, NUM_EXPERTS), axis=-1, keepdims=True)
    m2 = iota == i2
    denom = v1 + v2
    comb = jnp.where(m1, v1, 0.0) + jnp.where(m2, v2, 0.0)
    comb_ref[...] = jnp.transpose(comb / denom, (1, 0))
    hsh = _gelu(_dotf32(yb, ws1_ref[...]) + bs1_ref[...])
    shared = _dotf32(hsh.astype(BF), ws2_ref[...]) + bs2_ref[...]
    base_ref[...] = x2 + shared


def _k3(attn, xf, Wp_bf, b_proj, ln2_g, ln2_b, W_r, b_r, Ws1_bf, b_s1, Ws2_bf, b_s2):
    blk = 512
    return pl.pallas_call(
        _k3_body,
        grid=(T // blk,),
        in_specs=[
            pl.BlockSpec((blk, DIM), lambda i: (i, 0)),
            pl.BlockSpec((blk, DIM), lambda i: (i, 0)),
            pl.BlockSpec((DIM, DIM), lambda i: (0, 0)),
            pl.BlockSpec((DIM,), lambda i: (0,)),
            pl.BlockSpec((DIM,), lambda i: (0,)),
            pl.BlockSpec((DIM,), lambda i: (0,)),
            pl.BlockSpec((DIM, NUM_EXPERTS), lambda i: (0, 0)),
            pl.BlockSpec((NUM_EXPERTS,), lambda i: (0,)),
            pl.BlockSpec((DIM, HID), lambda i: (0, 0)),
            pl.BlockSpec((HID,), lambda i: (0,)),
            pl.BlockSpec((HID, DIM), lambda i: (0, 0)),
            pl.BlockSpec((DIM,), lambda i: (0,)),
        ],
        out_specs=[
            pl.BlockSpec((blk, DIM), lambda i: (i, 0)),
            pl.BlockSpec((blk, DIM), lambda i: (i, 0)),
            pl.BlockSpec((NUM_EXPERTS, blk), lambda i: (0, i)),
        ],
        out_shape=[
            jax.ShapeDtypeStruct((T, DIM), BF),
            jax.ShapeDtypeStruct((T, DIM), jnp.float32),
            jax.ShapeDtypeStruct((NUM_EXPERTS, T), jnp.float32),
        ],
    )(attn, xf, Wp_bf, b_proj, ln2_g, ln2_b, W_r, b_r, Ws1_bf, b_s1, Ws2_bf, b_s2)


def _k4_body(y_ref, comb_ref, w1_ref, b1_ref, w2_ref, b2_ref, base_ref, o_ref):
    e = pl.program_id(0)
    h = _gelu(_dotf32(y_ref[...], w1_ref[0]) + b1_ref[0])
    eo = _dotf32(h.astype(BF), w2_ref[0]) + b2_ref[0]
    contrib = comb_ref[0] * eo

    @pl.when(e == 0)
    def _():
        o_ref[...] = base_ref[...] + contrib

    @pl.when(e > 0)
    def _():
        o_ref[...] = o_ref[...] + contrib


def _k4(y_bf, comb3, We1_bf, b_e1, We2_bf, b_e2, base):
    return pl.pallas_call(
        _k4_body,
        grid=(NUM_EXPERTS,),
        in_specs=[
            pl.BlockSpec((T, DIM), lambda e: (0, 0)),
            pl.BlockSpec((1, T, 1), lambda e: (e, 0, 0)),
            pl.BlockSpec((1, DIM, HID), lambda e: (e, 0, 0)),
            pl.BlockSpec((1, 1, HID), lambda e: (e, 0, 0)),
            pl.BlockSpec((1, HID, DIM), lambda e: (e, 0, 0)),
            pl.BlockSpec((1, 1, DIM), lambda e: (e, 0, 0)),
            pl.BlockSpec((T, DIM), lambda e: (0, 0)),
        ],
        out_specs=pl.BlockSpec((T, DIM), lambda e: (0, 0)),
        out_shape=jax.ShapeDtypeStruct((T, DIM), jnp.float32),
    )(y_bf, comb3, We1_bf, b_e1, We2_bf, b_e2, base)


def kernel(x, ln1_g, ln1_b, ln2_g, ln2_b, W_qkv, b_qkv, rpb, W_proj, b_proj,
           W_r, b_r, W_e1, b_e1, W_e2, b_e2, W_s1, b_s1, W_s2, b_s2):
    xf = x.reshape(T, DIM)
    qkv = _k1(xf, ln1_g, ln1_b, W_qkv.astype(BF), b_qkv)
    qkv3 = qkv.reshape(B, H * W, 3 * DIM)
    bias = _make_bias(rpb)
    attn = _k2(bias, qkv3).reshape(T, DIM)
    y_bf, base, comb = _k3(attn, xf, W_proj.astype(BF), b_proj, ln2_g, ln2_b,
                           W_r, b_r, W_s1.astype(BF), b_s1, W_s2.astype(BF), b_s2)
    comb3 = comb.reshape(NUM_EXPERTS, T, 1)
    out = _k4(y_bf, comb3, W_e1.astype(BF), b_e1.reshape(NUM_EXPERTS, 1, HID),
              W_e2.astype(BF), b_e2.reshape(NUM_EXPERTS, 1, DIM), base)
    return out.reshape(B, H, W, DIM)
